# baseline (speedup 1.0000x reference)
import functools

import jax
import jax.numpy as jnp
from jax import lax
from jax.experimental import pallas as pl
from jax.experimental.pallas import tpu as pltpu
from jax.experimental.pallas import tpu_sc as plsc

DMODEL = 64
SCALE = 8.0
C = 128
NGB = 5
NTB = 5
NCOL = 200
VB = 128
RW = DMODEL + 1


def _make_transpose(nw, nc, V):
    nfull = V // VB
    vtail = V - nfull * VB
    nblk = nfull + 1
    mesh = plsc.VectorSubcoreMesh(core_axis_name="c", subcore_axis_name="s")
    NB = 3

    @functools.partial(
        pl.kernel,
        mesh=mesh,
        compiler_params=pltpu.CompilerParams(needs_layout_passes=False),
        out_type=jax.ShapeDtypeStruct((V * DMODEL,), jnp.float32),
        scratch_types=(
            [pltpu.VMEM((8, 8, VB), jnp.float32) for _ in range(NB)]
            + [pltpu.VMEM((VB, RW), jnp.float32) for _ in range(NB)]
            + [pltpu.VMEM((VB * DMODEL,), jnp.float32) for _ in range(NB)]
            + [pltpu.SemaphoreType.DMA for _ in range(2 * NB)]
        ),
    )
    def tr(t3_hbm, out_hbm, *rest):
        ibufs = rest[:NB]
        pbufs = rest[NB:2 * NB]
        obufs = rest[2 * NB:3 * NB]
        isems = rest[3 * NB:4 * NB]
        osems = rest[4 * NB:]
        wid = lax.axis_index("s") * nc + lax.axis_index("c")

        ci = lax.iota(jnp.int32, 16)
        zi = ci * 0
        rvk = [ci + 16 * k for k in range(VB // 16)]

        def start_in(b, s):
            @pl.when(b < nfull)
            def _():
                pltpu.async_copy(
                    t3_hbm.at[:, :, pl.ds(b * VB, VB)], ibufs[s], isems[s])

            @pl.when(b == nfull)
            def _():
                pltpu.async_copy(
                    t3_hbm.at[:, :, pl.ds(b * VB, vtail)],
                    ibufs[s].at[:, :, pl.ds(0, vtail)], isems[s])

        def wait_in(b, s):
            @pl.when(b < nfull)
            def _():
                pltpu.make_async_copy(
                    t3_hbm.at[:, :, pl.ds(0, VB)], ibufs[s], isems[s]).wait()

            @pl.when(b == nfull)
            def _():
                pltpu.make_async_copy(
                    t3_hbm.at[:, :, pl.ds(0, vtail)],
                    ibufs[s].at[:, :, pl.ds(0, vtail)], isems[s]).wait()

        def start_out(b, s):
            @pl.when(b < nfull)
            def _():
                pltpu.async_copy(
                    obufs[s], out_hbm.at[pl.ds(b * VB * DMODEL, VB * DMODEL)],
                    osems[s])

            @pl.when(b == nfull)
            def _():
                pltpu.async_copy(
                    obufs[s].at[pl.ds(0, vtail * DMODEL)],
                    out_hbm.at[pl.ds(b * VB * DMODEL, vtail * DMODEL)],
                    osems[s])

        def wait_out(b, s):
            @pl.when(b < nfull)
            def _():
                pltpu.make_async_copy(
                    obufs[s], out_hbm.at[pl.ds(0, VB * DMODEL)],
                    osems[s]).wait()

            @pl.when(b == nfull)
            def _():
                pltpu.make_async_copy(
                    obufs[s].at[pl.ds(0, vtail * DMODEL)],
                    out_hbm.at[pl.ds(0, vtail * DMODEL)], osems[s]).wait()

        def compute(b, s):
            ibuf, pbuf, obuf = ibufs[s], pbufs[s], obufs[s]

            def passes(nk):
                @plsc.parallel_loop(0, DMODEL, 1, unroll=2)
                def _cols(c):
                    cv = zi + c
                    for k in range(nk):
                        v = ibuf[c >> 3, c & 7, pl.ds(16 * k, 16)] * SCALE
                        plsc.store_scatter(pbuf, [rvk[k], cv], v)

                def _rows(i, acc):
                    rl = i * 2
                    for u in range(2):
                        for k4 in range(4):
                            obuf[pl.ds((rl + u) * DMODEL + 16 * k4, 16)] = (
                                pbuf[rl + u, pl.ds(16 * k4, 16)])
                    return acc
                lax.fori_loop(0, 8 * nk, _rows, 0)

            @pl.when(b < nfull)
            def _():
                passes(VB // 16)

            @pl.when(b == nfull)
            def _():
                passes(vtail // 16)

        nt = (nblk - wid + nw - 1) // nw

        for s in range(NB):
            @pl.when(s < nt)
            def _(s=s):
                start_in(wid + nw * s, s)

        def step(t, carry):
            for s in range(NB):
                tt = t * NB + s
                b = wid + nw * tt

                @pl.when(tt < nt)
                def _(s=s, tt=tt, b=b):
                    wait_in(b, s)

                    @pl.when(tt >= NB)
                    def _():
                        wait_out(b - nw * NB, s)

                    compute(b, s)

                    @pl.when(tt + NB < nt)
                    def _():
                        start_in(b + nw * NB, s)

                    start_out(b, s)
            return carry

        ntmax = (nblk + nw - 1) // nw
        lax.fori_loop(0, (ntmax + NB - 1) // NB, step, 0)

        for s in range(NB):
            @pl.when(s < nt)
            def _(s=s):
                t_last = nt - 1 - ((nt - 1 - s) % NB)
                wait_out(wid + nw * t_last, s)

    return tr


def _make_gather(nw, nc):
    mesh = plsc.VectorSubcoreMesh(core_axis_name="c", subcore_axis_name="s")

    @functools.partial(
        pl.kernel,
        mesh=mesh,
        compiler_params=pltpu.CompilerParams(
            use_tc_tiling_on_sc=False, needs_layout_passes=False),
        out_type=jax.ShapeDtypeStruct((NCOL, 8, nw, 8, C), jnp.float32),
        scratch_types=(
            [pltpu.VMEM((NCOL // 8, 1, 8, C), jnp.int32)]
            + [pltpu.VMEM((C, DMODEL), jnp.float32) for _ in range(NGB)]
            + [pltpu.VMEM((8, 1, 8, C + 1), jnp.float32) for _ in range(NTB)]
            + [pltpu.SemaphoreType.DMA for _ in range(NGB + NTB)]
        ),
    )
    def emb(x4_hbm, table_hbm, out_hbm, idxv, *rest):
        gbufs = rest[:NGB]
        tbufs = rest[NGB:NGB + NTB]
        gsems = rest[NGB + NTB:2 * NGB + NTB]
        osems = rest[2 * NGB + NTB:]
        wid = lax.axis_index("s") * nc + lax.axis_index("c")

        pltpu.sync_copy(x4_hbm.at[:, pl.ds(wid, 1)], idxv)

        ci = lax.iota(jnp.int32, 16)
        zi = ci * 0
        rowc = [(ci + 16 * q) >> 3 for q in range(4)]
        subc = [(ci + 16 * q) & 7 for q in range(4)]

        def compute(gbuf, tbuf):
            @plsc.parallel_loop(0, C, 1, unroll=4)
            def _rows(r):
                rv = zi + r
                for q in range(4):
                    v = gbuf[r, pl.ds(16 * q, 16)]
                    plsc.store_scatter(tbuf, [rowc[q], zi, subc[q], rv], v)

        def start_gather(j, g):
            pltpu.async_copy(
                table_hbm.at[idxv.at[j // 8, 0, j % 8]], gbufs[g], gsems[g])

        def wait_gather(g):
            pltpu.make_async_copy(
                table_hbm.at[idxv.at[0, 0, 0]], gbufs[g], gsems[g]).wait()

        def out_write(j, b):
            pltpu.async_copy(
                tbufs[b].at[:, :, :, pl.ds(0, C)],
                out_hbm.at[j, :, pl.ds(wid, 1)], osems[b])

        def out_wait(b):
            pltpu.make_async_copy(
                tbufs[b].at[:, :, :, pl.ds(0, C)],
                out_hbm.at[0, :, pl.ds(0, 1)], osems[b]).wait()

        for g in range(NGB):
            start_gather(g, g)

        def step(jj, carry):
            jb = jj * NGB
            for g in range(NGB):
                j = jb + g
                wait_gather(g)

                @pl.when(j >= NTB)
                def _(g=g):
                    out_wait(g)

                compute(gbufs[g], tbufs[g])

                @pl.when(j + NGB < NCOL)
                def _(g=g, j=j):
                    start_gather(j + NGB, g)

                out_write(j, g)
            return carry

        lax.fori_loop(0, NCOL // NGB, step, 0)

        for b in range(NTB):
            out_wait(b)

    return emb


def kernel(x, table):
    S, T = x.shape
    info = plsc.get_sparse_core_info()
    nc, ns = info.num_cores, info.num_subcores
    nw = nc * ns
    x4 = x.astype(jnp.int32).T.reshape(T // 8, 8, S // C, C).transpose(0, 2, 1, 3)
    V = table.shape[0]
    t3 = table.T.reshape(8, 8, V)
    tlin = _make_transpose(nw, nc, V)(t3).reshape(V, DMODEL)
    out5 = _make_gather(nw, nc)(x4, tlin)
    return out5.transpose(2, 4, 0, 1, 3).reshape(S, T, DMODEL)


# speedup vs baseline: 1.3355x; 1.3355x over previous
import functools

import jax
import jax.numpy as jnp
from jax import lax
from jax.experimental import pallas as pl
from jax.experimental.pallas import tpu as pltpu
from jax.experimental.pallas import tpu_sc as plsc

DMODEL = 64
SCALE = 8.0
C = 128
NGB = 5
NTB = 5
NCOL = 200
VB = 256
RW = DMODEL + 1


def _make_transpose(nw, nc, V):
    nfull = V // VB
    vtail = V - nfull * VB
    nblk = nfull + 1
    mesh = plsc.VectorSubcoreMesh(core_axis_name="c", subcore_axis_name="s")
    NB = 2

    @functools.partial(
        pl.kernel,
        mesh=mesh,
        compiler_params=pltpu.CompilerParams(needs_layout_passes=False),
        out_type=jax.ShapeDtypeStruct((V * DMODEL,), jnp.float32),
        scratch_types=(
            [pltpu.VMEM((8, 8, VB), jnp.float32) for _ in range(NB)]
            + [pltpu.VMEM((VB, RW), jnp.float32) for _ in range(NB)]
            + [pltpu.VMEM((VB * DMODEL,), jnp.float32) for _ in range(NB)]
            + [pltpu.SemaphoreType.DMA for _ in range(2 * NB)]
        ),
    )
    def tr(t3_hbm, out_hbm, *rest):
        ibufs = rest[:NB]
        pbufs = rest[NB:2 * NB]
        obufs = rest[2 * NB:3 * NB]
        isems = rest[3 * NB:4 * NB]
        osems = rest[4 * NB:]
        wid = lax.axis_index("s") * nc + lax.axis_index("c")

        ci = lax.iota(jnp.int32, 16)
        zi = ci * 0
        rvk = [ci + 16 * k for k in range(VB // 16)]

        def start_in(b, s):
            @pl.when(b < nfull)
            def _():
                pltpu.async_copy(
                    t3_hbm.at[:, :, pl.ds(b * VB, VB)], ibufs[s], isems[s])

            @pl.when(b == nfull)
            def _():
                pltpu.async_copy(
                    t3_hbm.at[:, :, pl.ds(b * VB, vtail)],
                    ibufs[s].at[:, :, pl.ds(0, vtail)], isems[s])

        def wait_in(b, s):
            @pl.when(b < nfull)
            def _():
                pltpu.make_async_copy(
                    t3_hbm.at[:, :, pl.ds(0, VB)], ibufs[s], isems[s]).wait()

            @pl.when(b == nfull)
            def _():
                pltpu.make_async_copy(
                    t3_hbm.at[:, :, pl.ds(0, vtail)],
                    ibufs[s].at[:, :, pl.ds(0, vtail)], isems[s]).wait()

        def start_out(b, s):
            @pl.when(b < nfull)
            def _():
                pltpu.async_copy(
                    obufs[s], out_hbm.at[pl.ds(b * VB * DMODEL, VB * DMODEL)],
                    osems[s])

            @pl.when(b == nfull)
            def _():
                pltpu.async_copy(
                    obufs[s].at[pl.ds(0, vtail * DMODEL)],
                    out_hbm.at[pl.ds(b * VB * DMODEL, vtail * DMODEL)],
                    osems[s])

        def wait_out(b, s):
            @pl.when(b < nfull)
            def _():
                pltpu.make_async_copy(
                    obufs[s], out_hbm.at[pl.ds(0, VB * DMODEL)],
                    osems[s]).wait()

            @pl.when(b == nfull)
            def _():
                pltpu.make_async_copy(
                    obufs[s].at[pl.ds(0, vtail * DMODEL)],
                    out_hbm.at[pl.ds(0, vtail * DMODEL)], osems[s]).wait()

        def compute(b, s):
            ibuf, pbuf, obuf = ibufs[s], pbufs[s], obufs[s]

            def passes(nk):
                @plsc.parallel_loop(0, DMODEL, 1, unroll=2)
                def _cols(c):
                    cv = zi + c
                    for k in range(nk):
                        v = ibuf[c >> 3, c & 7, pl.ds(16 * k, 16)] * SCALE
                        plsc.store_scatter(pbuf, [rvk[k], cv], v)

                @plsc.parallel_loop(0, 16 * nk, 1, unroll=4)
                def _rows(rl):
                    for k4 in range(4):
                        obuf[pl.ds(rl * DMODEL + 16 * k4, 16)] = (
                            pbuf[rl, pl.ds(16 * k4, 16)])

            @pl.when(b < nfull)
            def _():
                passes(VB // 16)

            @pl.when(b == nfull)
            def _():
                passes(vtail // 16)

        nt = (nblk - wid + nw - 1) // nw

        for s in range(NB):
            @pl.when(s < nt)
            def _(s=s):
                start_in(wid + nw * s, s)

        def step(t, carry):
            for s in range(NB):
                tt = t * NB + s
                b = wid + nw * tt

                @pl.when(tt < nt)
                def _(s=s, tt=tt, b=b):
                    wait_in(b, s)

                    @pl.when(tt >= NB)
                    def _():
                        wait_out(b - nw * NB, s)

                    compute(b, s)

                    @pl.when(tt + NB < nt)
                    def _():
                        start_in(b + nw * NB, s)

                    start_out(b, s)
            return carry

        ntmax = (nblk + nw - 1) // nw
        lax.fori_loop(0, (ntmax + NB - 1) // NB, step, 0)

        for s in range(NB):
            @pl.when(s < nt)
            def _(s=s):
                t_last = nt - 1 - ((nt - 1 - s) % NB)
                wait_out(wid + nw * t_last, s)

    return tr


def _make_gather(nw, nc):
    mesh = plsc.VectorSubcoreMesh(core_axis_name="c", subcore_axis_name="s")

    @functools.partial(
        pl.kernel,
        mesh=mesh,
        compiler_params=pltpu.CompilerParams(
            use_tc_tiling_on_sc=False, needs_layout_passes=False),
        out_type=jax.ShapeDtypeStruct((NCOL, 8, nw, 8, C), jnp.float32),
        scratch_types=(
            [pltpu.VMEM((NCOL // 8, 1, 8, C), jnp.int32)]
            + [pltpu.VMEM((C, DMODEL), jnp.float32) for _ in range(NGB)]
            + [pltpu.VMEM((8, 1, 8, C + 1), jnp.float32) for _ in range(NTB)]
            + [pltpu.SemaphoreType.DMA for _ in range(NGB + NTB)]
        ),
    )
    def emb(x4_hbm, table_hbm, out_hbm, idxv, *rest):
        gbufs = rest[:NGB]
        tbufs = rest[NGB:NGB + NTB]
        gsems = rest[NGB + NTB:2 * NGB + NTB]
        osems = rest[2 * NGB + NTB:]
        wid = lax.axis_index("s") * nc + lax.axis_index("c")

        pltpu.sync_copy(x4_hbm.at[:, pl.ds(wid, 1)], idxv)

        ci = lax.iota(jnp.int32, 16)
        zi = ci * 0
        rowc = [(ci + 16 * q) >> 3 for q in range(4)]
        subc = [(ci + 16 * q) & 7 for q in range(4)]

        def compute(gbuf, tbuf):
            @plsc.parallel_loop(0, C, 1, unroll=4)
            def _rows(r):
                rv = zi + r
                for q in range(4):
                    v = gbuf[r, pl.ds(16 * q, 16)]
                    plsc.store_scatter(tbuf, [rowc[q], zi, subc[q], rv], v)

        def start_gather(j, g):
            pltpu.async_copy(
                table_hbm.at[idxv.at[j // 8, 0, j % 8]], gbufs[g], gsems[g])

        def wait_gather(g):
            pltpu.make_async_copy(
                table_hbm.at[idxv.at[0, 0, 0]], gbufs[g], gsems[g]).wait()

        def out_write(j, b):
            pltpu.async_copy(
                tbufs[b].at[:, :, :, pl.ds(0, C)],
                out_hbm.at[j, :, pl.ds(wid, 1)], osems[b])

        def out_wait(b):
            pltpu.make_async_copy(
                tbufs[b].at[:, :, :, pl.ds(0, C)],
                out_hbm.at[0, :, pl.ds(0, 1)], osems[b]).wait()

        for g in range(NGB):
            start_gather(g, g)

        def step(jj, carry):
            jb = jj * NGB
            for g in range(NGB):
                j = jb + g
                wait_gather(g)

                @pl.when(j >= NTB)
                def _(g=g):
                    out_wait(g)

                compute(gbufs[g], tbufs[g])

                @pl.when(j + NGB < NCOL)
                def _(g=g, j=j):
                    start_gather(j + NGB, g)

                out_write(j, g)
            return carry

        lax.fori_loop(0, NCOL // NGB, step, 0)

        for b in range(NTB):
            out_wait(b)

    return emb


def kernel(x, table):
    S, T = x.shape
    info = plsc.get_sparse_core_info()
    nc, ns = info.num_cores, info.num_subcores
    nw = nc * ns
    x4 = x.astype(jnp.int32).T.reshape(T // 8, 8, S // C, C).transpose(0, 2, 1, 3)
    V = table.shape[0]
    t3 = table.T.reshape(8, 8, V)
    tlin = _make_transpose(nw, nc, V)(t3).reshape(V, DMODEL)
    out5 = _make_gather(nw, nc)(x4, tlin)
    return out5.transpose(2, 4, 0, 1, 3).reshape(S, T, DMODEL)


# speedup vs baseline: 1.9514x; 1.4612x over previous
import functools

import jax
import jax.numpy as jnp
from jax import lax
from jax.experimental import pallas as pl
from jax.experimental.pallas import tpu as pltpu
from jax.experimental.pallas import tpu_sc as plsc

DMODEL = 64
SCALE = 8.0
C = 128
NGB = 5
NTB = 5
NCOL = 200


def _make_gather(nw, nc):
    mesh = plsc.VectorSubcoreMesh(core_axis_name="c", subcore_axis_name="s")

    @functools.partial(
        pl.kernel,
        mesh=mesh,
        compiler_params=pltpu.CompilerParams(
            use_tc_tiling_on_sc=False, needs_layout_passes=False),
        out_type=jax.ShapeDtypeStruct((NCOL, 8, nw, 8, C), jnp.float32),
        scratch_types=(
            [pltpu.VMEM((NCOL // 8, 1, 8, C), jnp.int32)]
            + [pltpu.VMEM((C, DMODEL), jnp.float32) for _ in range(NGB)]
            + [pltpu.VMEM((8, 1, 8, C + 1), jnp.float32) for _ in range(NTB)]
            + [pltpu.SemaphoreType.DMA for _ in range(NGB + NTB)]
        ),
    )
    def emb(x4_hbm, table_hbm, out_hbm, idxv, *rest):
        gbufs = rest[:NGB]
        tbufs = rest[NGB:NGB + NTB]
        gsems = rest[NGB + NTB:2 * NGB + NTB]
        osems = rest[2 * NGB + NTB:]
        wid = lax.axis_index("s") * nc + lax.axis_index("c")

        pltpu.sync_copy(x4_hbm.at[:, pl.ds(wid, 1)], idxv)

        ci = lax.iota(jnp.int32, 16)
        zi = ci * 0
        rowc = [(ci + 16 * q) >> 3 for q in range(4)]
        subc = [(ci + 16 * q) & 7 for q in range(4)]

        def compute(gbuf, tbuf):
            @plsc.parallel_loop(0, C, 1, unroll=4)
            def _rows(r):
                rv = zi + r
                for q in range(4):
                    v = gbuf[r, pl.ds(16 * q, 16)] * SCALE
                    plsc.store_scatter(tbuf, [rowc[q], zi, subc[q], rv], v)

        def start_gather(j, g):
            pltpu.async_copy(
                table_hbm.at[idxv.at[j // 8, 0, j % 8]], gbufs[g], gsems[g])

        def wait_gather(g):
            pltpu.make_async_copy(
                table_hbm.at[idxv.at[0, 0, 0]], gbufs[g], gsems[g]).wait()

        def out_write(j, b):
            pltpu.async_copy(
                tbufs[b].at[:, :, :, pl.ds(0, C)],
                out_hbm.at[j, :, pl.ds(wid, 1)], osems[b])

        def out_wait(b):
            pltpu.make_async_copy(
                tbufs[b].at[:, :, :, pl.ds(0, C)],
                out_hbm.at[0, :, pl.ds(0, 1)], osems[b]).wait()

        for g in range(NGB):
            start_gather(g, g)

        def step(jj, carry):
            jb = jj * NGB
            for g in range(NGB):
                j = jb + g
                wait_gather(g)

                @pl.when(j >= NTB)
                def _(g=g):
                    out_wait(g)

                compute(gbufs[g], tbufs[g])

                @pl.when(j + NGB < NCOL)
                def _(g=g, j=j):
                    start_gather(j + NGB, g)

                out_write(j, g)
            return carry

        lax.fori_loop(0, NCOL // NGB, step, 0)

        for b in range(NTB):
            out_wait(b)

    return emb


def kernel(x, table):
    S, T = x.shape
    info = plsc.get_sparse_core_info()
    nc, ns = info.num_cores, info.num_subcores
    nw = nc * ns
    x4 = x.astype(jnp.int32).T.reshape(T // 8, 8, S // C, C).transpose(0, 2, 1, 3)
    out5 = _make_gather(nw, nc)(x4, table)
    return out5.transpose(2, 4, 0, 1, 3).reshape(S, T, DMODEL)


# speedup vs baseline: 1.9545x; 1.0016x over previous
import functools

import jax
import jax.numpy as jnp
from jax import lax
from jax.experimental import pallas as pl
from jax.experimental.pallas import tpu as pltpu
from jax.experimental.pallas import tpu_sc as plsc

DMODEL = 64
SCALE = 8.0
C = 128
NGB = 5
NTB = 5
NCOL = 200


def _make_gather(nw, nc):
    mesh = plsc.VectorSubcoreMesh(core_axis_name="c", subcore_axis_name="s")

    @functools.partial(
        pl.kernel,
        mesh=mesh,
        compiler_params=pltpu.CompilerParams(
            use_tc_tiling_on_sc=False, needs_layout_passes=False),
        out_type=jax.ShapeDtypeStruct((NCOL, 8, nw, 8, C), jnp.float32),
        scratch_types=(
            [pltpu.VMEM((NCOL // 8, 1, 8, C), jnp.int32)]
            + [pltpu.VMEM((C, DMODEL), jnp.float32) for _ in range(NGB)]
            + [pltpu.VMEM((8, 1, 8, C + 1), jnp.float32) for _ in range(NTB)]
            + [pltpu.SemaphoreType.DMA for _ in range(NGB + NTB)]
        ),
    )
    def emb(x4_hbm, table_hbm, out_hbm, idxv, *rest):
        gbufs = rest[:NGB]
        tbufs = rest[NGB:NGB + NTB]
        gsems = rest[NGB + NTB:2 * NGB + NTB]
        osems = rest[2 * NGB + NTB:]
        wid = lax.axis_index("s") * nc + lax.axis_index("c")

        pltpu.sync_copy(x4_hbm.at[:, pl.ds(wid, 1)], idxv)

        ci = lax.iota(jnp.int32, 16)
        zi = ci * 0
        rowc = [(ci + 16 * q) >> 3 for q in range(4)]
        subc = [(ci + 16 * q) & 7 for q in range(4)]

        def compute(gbuf, tbuf):
            @plsc.parallel_loop(0, C, 1, unroll=4)
            def _rows(r):
                rv = zi + r
                for q in range(4):
                    v = gbuf[r, pl.ds(16 * q, 16)] * SCALE
                    plsc.store_scatter(tbuf, [rowc[q], zi, subc[q], rv], v)

        def start_gather(j, g):
            pltpu.async_copy(
                table_hbm.at[idxv.at[j // 8, 0, j % 8]], gbufs[g], gsems[g])

        def wait_gather(g):
            pltpu.make_async_copy(
                table_hbm.at[idxv.at[0, 0, 0]], gbufs[g], gsems[g]).wait()

        def out_write(j, b):
            pltpu.async_copy(
                tbufs[b].at[:, :, :, pl.ds(0, C)],
                out_hbm.at[j, :, pl.ds(wid, 1)], osems[b])

        def out_wait(b):
            pltpu.make_async_copy(
                tbufs[b].at[:, :, :, pl.ds(0, C)],
                out_hbm.at[0, :, pl.ds(0, 1)], osems[b]).wait()

        for g in range(NGB):
            start_gather(g, g)

        def step(jj, carry):
            jb = jj * NGB
            for g in range(NGB):
                j = jb + g
                wait_gather(g)

                @pl.when(j >= 1)
                def _(g=g, j=j):
                    out_write(j - 1, (g - 1) % NGB)

                @pl.when(j >= NTB)
                def _(g=g):
                    out_wait(g)

                compute(gbufs[g], tbufs[g])

                @pl.when(j + NGB < NCOL)
                def _(g=g, j=j):
                    start_gather(j + NGB, g)
            return carry

        lax.fori_loop(0, NCOL // NGB, step, 0)

        out_write(NCOL - 1, NGB - 1)
        for b in range(NTB):
            out_wait(b)

    return emb


def kernel(x, table):
    S, T = x.shape
    info = plsc.get_sparse_core_info()
    nc, ns = info.num_cores, info.num_subcores
    nw = nc * ns
    x4 = x.astype(jnp.int32).T.reshape(T // 8, 8, S // C, C).transpose(0, 2, 1, 3)
    out5 = _make_gather(nw, nc)(x4, table)
    return out5.transpose(2, 4, 0, 1, 3).reshape(S, T, DMODEL)
